# Initial kernel scaffold; baseline (speedup 1.0000x reference)
import jax
import jax.numpy as jnp
from jax.experimental import pallas as pl


def kernel(x, edge_index, W1, W2, lstm1_W, lstm1_b, lstm2_W, lstm2_b, W_out, b_out):
    raise NotImplementedError("write your pallas kernel here")


# speedup vs baseline: 5.4081x; 5.4081x over previous
import jax
import jax.numpy as jnp
from jax import lax
from jax.experimental import pallas as pl
from jax.experimental.pallas import tpu as pltpu
from jax.experimental.pallas import tpu_sc as plsc

_NC = 2
_NS = 16
_NW = _NC * _NS
_CH = 128


def _acc_rows(n):
  return ((n + _NS * _CH - 1) // (_NS * _CH)) * (_NS * _CH)


def _sc_mesh():
  return plsc.VectorSubcoreMesh(
      core_axis_name="c", subcore_axis_name="s",
      num_cores=_NC, num_subcores=_NS)


def _chunking(n, e, d):
  assert e % _NW == 0
  e_w = e // _NW
  assert e_w % _CH == 0
  n_ch = e_w // _CH
  na = _acc_rows(n)
  rows_w = na // _NS
  assert rows_w % _CH == 0 and d % 16 == 0
  nz = rows_w // _CH
  return e_w, n_ch, na, rows_w, nz


def _make_scatter_pass(n, e, d):
  e_w, n_ch, na, rows_w, nz = _chunking(n, e, d)

  @pl.kernel(
      out_type=jax.ShapeDtypeStruct((_NC, na, d), jnp.float32),
      mesh=_sc_mesh(),
      scratch_types=(
          pltpu.VMEM((_CH,), jnp.int32),
          pltpu.VMEM((_CH,), jnp.int32),
          pltpu.VMEM((_CH,), jnp.int32),
          pltpu.VMEM((_CH, d), jnp.float32),
          pltpu.VMEM_SHARED((na, d), jnp.float32),
          pltpu.SemaphoreType.DMA,
      ),
  )
  def sc_pass(x_hbm, src_hbm, dst_hbm, acc_out, src_v, dst_v, idx_v, rows_v,
              acc, sem):
    cid = lax.axis_index("c")
    sid = lax.axis_index("s")
    row0 = sid * rows_w
    i16 = lax.iota(jnp.int32, 16)
    z16 = jnp.zeros((16,), jnp.float32)

    def fill_idx(start):
      for k in range(_CH // 16):
        idx_v[pl.ds(k * 16, 16)] = i16 + (start + k * 16)

    def zfill(r, carry):
      for k in range(d // 16):
        rows_v[r, pl.ds(k * 16, 16)] = z16
      return carry
    lax.fori_loop(0, _CH, zfill, 0)

    def zblock(b, carry):
      fill_idx(row0 + b * _CH)
      pltpu.sync_copy(rows_v, acc.at[idx_v])
      return carry
    lax.fori_loop(0, nz, zblock, 0)
    plsc.subcore_barrier()

    base = (sid * _NC + cid) * e_w
    def edge_body(j, carry):
      off = base + j * _CH
      pltpu.sync_copy(src_hbm.at[pl.ds(off, _CH)], src_v)
      pltpu.sync_copy(dst_hbm.at[pl.ds(off, _CH)], dst_v)
      pltpu.async_copy(x_hbm.at[src_v], rows_v, sem).wait()
      pltpu.sync_copy(rows_v, acc.at[dst_v], add=True)
      return carry
    lax.fori_loop(0, n_ch, edge_body, 0)
    plsc.subcore_barrier()

    def oblock(b, carry):
      fill_idx(row0 + b * _CH)
      pltpu.async_copy(acc.at[idx_v], rows_v, sem).wait()
      pltpu.sync_copy(rows_v, acc_out.at[cid, pl.ds(row0 + b * _CH, _CH)])
      return carry
    lax.fori_loop(0, nz, oblock, 0)

  return sc_pass


def _make_deg_pass(n, e):
  d = 128
  e_w, n_ch, na, rows_w, nz = _chunking(n, e, d)

  @pl.kernel(
      out_type=jax.ShapeDtypeStruct((_NC, na, d), jnp.float32),
      mesh=_sc_mesh(),
      scratch_types=(
          pltpu.VMEM((_CH,), jnp.int32),
          pltpu.VMEM((_CH,), jnp.int32),
          pltpu.VMEM((_CH, d), jnp.float32),
          pltpu.VMEM_SHARED((na, d), jnp.float32),
          pltpu.SemaphoreType.DMA,
      ),
  )
  def deg_pass(dst_hbm, deg_out, dst_v, idx_v, rows_v, acc, sem):
    cid = lax.axis_index("c")
    sid = lax.axis_index("s")
    row0 = sid * rows_w
    i16 = lax.iota(jnp.int32, 16)

    def fill_idx(start):
      for k in range(_CH // 16):
        idx_v[pl.ds(k * 16, 16)] = i16 + (start + k * 16)

    def fill_rows(val):
      v16 = jnp.full((16,), val, jnp.float32)
      def rfill(r, carry):
        for k in range(d // 16):
          rows_v[r, pl.ds(k * 16, 16)] = v16
        return carry
      lax.fori_loop(0, _CH, rfill, 0)

    fill_rows(0.0)
    def zblock(b, carry):
      fill_idx(row0 + b * _CH)
      pltpu.sync_copy(rows_v, acc.at[idx_v])
      return carry
    lax.fori_loop(0, nz, zblock, 0)
    fill_rows(1.0)
    plsc.subcore_barrier()

    base = (sid * _NC + cid) * e_w
    def edge_body(j, carry):
      off = base + j * _CH
      pltpu.sync_copy(dst_hbm.at[pl.ds(off, _CH)], dst_v)
      pltpu.sync_copy(rows_v, acc.at[dst_v], add=True)
      return carry
    lax.fori_loop(0, n_ch, edge_body, 0)
    plsc.subcore_barrier()

    def oblock(b, carry):
      fill_idx(row0 + b * _CH)
      pltpu.async_copy(acc.at[idx_v], rows_v, sem).wait()
      pltpu.sync_copy(rows_v, deg_out.at[cid, pl.ds(row0 + b * _CH, _CH)])
      return carry
    lax.fori_loop(0, nz, oblock, 0)

  return deg_pass


def _sigmoid(z):
  return 1.0 / (1.0 + jnp.exp(-z))


def _stage_a(n, p, degp, w1):
  d = p.shape[2]
  br = 400
  assert n % br == 0

  def body(p_ref, deg_ref, w_ref, h_ref, invd_ref):
    m = p_ref[0] + p_ref[1]
    deg = deg_ref[0][:, 0:1] + deg_ref[1][:, 0:1]
    invd = 1.0 / jnp.maximum(deg, 1.0)
    h = jnp.dot(m * invd, w_ref[...], preferred_element_type=jnp.float32)
    h_ref[...] = jnp.maximum(h, 0.0)
    invd_ref[...] = invd

  return pl.pallas_call(
      body,
      grid=(n // br,),
      in_specs=[
          pl.BlockSpec((2, br, d), lambda i: (0, i, 0)),
          pl.BlockSpec((2, br, d), lambda i: (0, i, 0)),
          pl.BlockSpec((d, d), lambda i: (0, 0)),
      ],
      out_specs=[
          pl.BlockSpec((br, d), lambda i: (i, 0)),
          pl.BlockSpec((br, 1), lambda i: (i, 0)),
      ],
      out_shape=[
          jax.ShapeDtypeStruct((n, d), jnp.float32),
          jax.ShapeDtypeStruct((n, 1), jnp.float32),
      ],
  )(p, degp, w1)


def _stage_b(n, q, invd, w2, g1, b1, g2, b2, wo, bo):
  d = q.shape[2]
  br = 400
  assert n % br == 0

  def body(q_ref, invd_ref, w2_ref, g1_ref, b1_ref, g2_ref, b2_ref, wo_ref,
           bo_ref, o_ref):
    m = q_ref[0] + q_ref[1]
    a = jnp.dot(m * invd_ref[...], w2_ref[...],
                preferred_element_type=jnp.float32)
    z1 = jnp.dot(a, g1_ref[...], preferred_element_type=jnp.float32) + b1_ref[...]
    h1 = _sigmoid(z1[:, 2 * d:]) * jnp.tanh(
        _sigmoid(z1[:, :d]) * jnp.tanh(z1[:, d:2 * d]))
    z2 = jnp.dot(h1, g2_ref[...], preferred_element_type=jnp.float32) + b2_ref[...]
    h2 = _sigmoid(z2[:, 2 * d:]) * jnp.tanh(
        _sigmoid(z2[:, :d]) * jnp.tanh(z2[:, d:2 * d]))
    out = jnp.sum(h2 * wo_ref[...], axis=1, keepdims=True) + bo_ref[0, 0]
    o_ref[...] = out

  return pl.pallas_call(
      body,
      grid=(n // br,),
      in_specs=[
          pl.BlockSpec((2, br, d), lambda i: (0, i, 0)),
          pl.BlockSpec((br, 1), lambda i: (i, 0)),
          pl.BlockSpec((d, d), lambda i: (0, 0)),
          pl.BlockSpec((d, 3 * d), lambda i: (0, 0)),
          pl.BlockSpec((1, 3 * d), lambda i: (0, 0)),
          pl.BlockSpec((d, 3 * d), lambda i: (0, 0)),
          pl.BlockSpec((1, 3 * d), lambda i: (0, 0)),
          pl.BlockSpec((1, d), lambda i: (0, 0)),
          pl.BlockSpec((1, 1), lambda i: (0, 0)),
      ],
      out_specs=pl.BlockSpec((br, 1), lambda i: (i, 0)),
      out_shape=jax.ShapeDtypeStruct((n, 1), jnp.float32),
  )(q, invd, w2, g1, b1, g2, b2, wo, bo)


def kernel(x, edge_index, W1, W2, lstm1_W, lstm1_b, lstm2_W, lstm2_b, W_out,
           b_out):
  n, d = x.shape
  e = edge_index.shape[1]
  h = W1.shape[1]
  na = _acc_rows(n)

  e_pad = ((e + _NW * _CH - 1) // (_NW * _CH)) * (_NW * _CH)
  src = edge_index[0]
  dst = edge_index[1]
  if e_pad != e:
    src = jnp.concatenate([src, jnp.zeros((e_pad - e,), jnp.int32)])
    dst = jnp.concatenate([dst, jnp.full((e_pad - e,), na - 1, jnp.int32)])

  def pack_gates(wg, bg):
    gm = jnp.concatenate([wg[0, :h], wg[2, :h], wg[3, :h]], axis=1)
    gb = jnp.concatenate([bg[0], bg[2], bg[3]])[None, :]
    return gm, gb

  g1, b1 = pack_gates(lstm1_W, lstm1_b)
  g2, b2 = pack_gates(lstm2_W, lstm2_b)
  wo = W_out.T
  bo = b_out.reshape(1, 1)

  deg_pass = _make_deg_pass(n, e_pad)
  pass1 = _make_scatter_pass(n, e_pad, d)
  pass2 = _make_scatter_pass(n, e_pad, h)

  degp = deg_pass(dst)
  p = pass1(x, src, dst)
  h1, invd = _stage_a(n, p, degp, W1)
  q = pass2(h1, src, dst)
  return _stage_b(n, q, invd, W2, g1, b1, g2, b2, wo, bo)
